# Initial kernel scaffold; baseline (speedup 1.0000x reference)
import functools

import jax
import jax.numpy as jnp
from jax import lax
from jax.experimental import pallas as pl
from jax.experimental.pallas import tpu as pltpu
from jax.experimental.pallas import tpu_sc as plsc

N = 10000
D = 128
E = 320000
NC = 2
NS = 16
NW = NC * NS
EPT = E // NW
K = 80
NCHUNK = EPT // K
RPT = N // NS
R = 400
GRID = N // R

_mesh = plsc.VectorSubcoreMesh(core_axis_name="c", subcore_axis_name="s")


@functools.partial(
    pl.kernel,
    out_type=jax.ShapeDtypeStruct((NC, N, 16), jnp.float32),
    mesh=_mesh,
    scratch_types=[
        pltpu.VMEM((K, 16), jnp.float32),
        pltpu.VMEM((K,), jnp.int32),
        pltpu.VMEM((K,), jnp.int32),
        pltpu.VMEM_SHARED((N, 16), jnp.float32),
        pltpu.SemaphoreType.DMA,
        pltpu.SemaphoreType.DMA,
    ],
)
def _deg_kernel(dst_hbm, zrows_hbm, out_hbm, ones_v, idx0, idx1, acc, sem0, sem1):
    cid = lax.axis_index("c")
    sid = lax.axis_index("s")
    wid = sid * NC + cid
    base = wid * EPT
    idxb = (idx0, idx1)
    semb = (sem0, sem1)

    def fill(i, carry):
        ones_v[i, :] = jnp.ones((16,), jnp.float32)
        return carry

    lax.fori_loop(0, K, fill, 0)
    pltpu.sync_copy(zrows_hbm, acc.at[pl.ds(sid * RPT, RPT)])
    plsc.subcore_barrier()

    def _cp(c, b):
        off = pl.multiple_of(base + c * K, 8)
        return pltpu.make_async_copy(dst_hbm.at[pl.ds(off, K)], idxb[b], semb[b])

    def scatter(b):
        pltpu.sync_copy(ones_v, acc.at[idxb[b]], add=True)

    _cp(0, 0).start()

    def pair(j, carry):
        a = 2 * j
        _cp(a + 1, 1).start()
        _cp(a, 0).wait()
        scatter(0)
        _cp(a + 2, 0).start()
        _cp(a + 1, 1).wait()
        scatter(1)
        return carry

    lax.fori_loop(0, (NCHUNK - 1) // 2, pair, 0)
    _cp(NCHUNK - 1, 0).wait()
    scatter(0)

    plsc.subcore_barrier()
    pltpu.sync_copy(acc.at[pl.ds(sid * RPT, RPT)],
                    out_hbm.at[cid, pl.ds(sid * RPT, RPT)])


@functools.partial(
    pl.kernel,
    out_type=jax.ShapeDtypeStruct((NC, N, D), jnp.float32),
    mesh=_mesh,
    scratch_types=[
        pltpu.VMEM((K,), jnp.int32),
        pltpu.VMEM((K,), jnp.int32),
        pltpu.VMEM((K,), jnp.int32),
        pltpu.VMEM((K,), jnp.int32),
        pltpu.VMEM((K, D), jnp.float32),
        pltpu.VMEM((K, D), jnp.float32),
        pltpu.VMEM_SHARED((N, D), jnp.float32),
        pltpu.SemaphoreType.DMA,
        pltpu.SemaphoreType.DMA,
    ],
)
def _agg_kernel(hp_hbm, src_hbm, dst_hbm, zrows_hbm, out_hbm,
                s0, s1, d0, d1, r0, r1, acc, g0, g1):
    cid = lax.axis_index("c")
    sid = lax.axis_index("s")
    wid = sid * NC + cid
    base = wid * EPT
    sb = (s0, s1)
    db = (d0, d1)
    rb = (r0, r1)
    gsem = (g0, g1)

    pltpu.sync_copy(zrows_hbm, acc.at[pl.ds(sid * RPT, RPT)])
    plsc.subcore_barrier()

    def start(c, b):
        off = pl.multiple_of(base + c * K, 8)
        pltpu.sync_copy(src_hbm.at[pl.ds(off, K)], sb[b])
        pltpu.sync_copy(dst_hbm.at[pl.ds(off, K)], db[b])
        pltpu.make_async_copy(hp_hbm.at[sb[b]], rb[b], gsem[b]).start()

    def drain(b):
        pltpu.make_async_copy(hp_hbm.at[sb[b]], rb[b], gsem[b]).wait()
        pltpu.sync_copy(rb[b], acc.at[db[b]], add=True)

    start(0, 0)

    def pair(j, carry):
        a = 2 * j
        start(a + 1, 1)
        drain(0)
        start(a + 2, 0)
        drain(1)
        return carry

    lax.fori_loop(0, (NCHUNK - 1) // 2, pair, 0)
    drain(0)

    plsc.subcore_barrier()
    pltpu.sync_copy(acc.at[pl.ds(sid * RPT, RPT)],
                    out_hbm.at[cid, pl.ds(sid * RPT, RPT)])


def _dinv_from(deg_blk):
    deg = deg_blk[:, 0] + deg_blk[:, 16] + 1.0
    return lax.rsqrt(deg)[:, None]


def _tc1_body(x_ref, w1_ref, deg_ref, o_ref):
    dinv = _dinv_from(deg_ref[...])
    h = jnp.dot(x_ref[...], w1_ref[...], preferred_element_type=jnp.float32)
    o_ref[...] = h * dinv


def _tc2_body(agg_ref, hp_ref, deg_ref, b1_ref, w2_ref, o_ref):
    dinv = _dinv_from(deg_ref[...])
    s = agg_ref[0] + agg_ref[1] + hp_ref[...]
    z = jnp.tanh(dinv * s + b1_ref[...])
    o_ref[...] = dinv * jnp.dot(z, w2_ref[...], preferred_element_type=jnp.float32)


def _tc3_body(agg_ref, hp_ref, deg_ref, b2_ref, wf1_ref, bf1_ref, wf2_ref,
              bf2_ref, o_ref):
    dinv = _dinv_from(deg_ref[...])
    s = agg_ref[0] + agg_ref[1] + hp_ref[...]
    z = jnp.tanh(dinv * s + b2_ref[...])
    f = jnp.tanh(jnp.dot(z, wf1_ref[...], preferred_element_type=jnp.float32)
                 + bf1_ref[...])
    o_ref[...] = (jnp.dot(f, wf2_ref[...], preferred_element_type=jnp.float32)
                  + bf2_ref[...])


def _row_spec(width):
    return pl.BlockSpec((R, width), lambda i: (i, 0))


def _full(shape):
    return pl.BlockSpec(shape, lambda i, _s=shape: tuple(0 for _ in _s))


_agg_spec = pl.BlockSpec((NC, R, D), lambda i: (0, i, 0))

_tc1 = pl.pallas_call(
    _tc1_body,
    grid=(GRID,),
    in_specs=[_row_spec(D), _full((D, D)), _row_spec(32)],
    out_specs=_row_spec(D),
    out_shape=jax.ShapeDtypeStruct((N, D), jnp.float32),
)

_tc2 = pl.pallas_call(
    _tc2_body,
    grid=(GRID,),
    in_specs=[_agg_spec, _row_spec(D), _row_spec(32), _full((1, D)),
              _full((D, D))],
    out_specs=_row_spec(D),
    out_shape=jax.ShapeDtypeStruct((N, D), jnp.float32),
)

_tc3 = pl.pallas_call(
    _tc3_body,
    grid=(GRID,),
    in_specs=[_agg_spec, _row_spec(D), _row_spec(32), _full((1, D)),
              _full((D, 64)), _full((1, 64)), _full((64, 1)), _full((1, 1))],
    out_specs=_row_spec(1),
    out_shape=jax.ShapeDtypeStruct((N, 1), jnp.float32),
)


def kernel(x, edge_index, W1, b1, W2, b2, Wf1, bf1, Wf2, bf2):
    ei = edge_index.astype(jnp.int32)
    src = ei[0]
    dst = ei[1]
    z16 = jnp.zeros((RPT, 16), jnp.float32)
    z128 = jnp.zeros((RPT, D), jnp.float32)

    degp = _deg_kernel(dst, z16)
    deg2 = jnp.transpose(degp, (1, 0, 2)).reshape(N, 32)

    h1p = _tc1(x, W1, deg2)
    agg1 = _agg_kernel(h1p, src, dst, z128)
    h2p = _tc2(agg1, h1p, deg2, b1.reshape(1, D), W2)
    agg2 = _agg_kernel(h2p, src, dst, z128)
    out = _tc3(agg2, h2p, deg2, b2.reshape(1, D), Wf1, bf1.reshape(1, 64),
               Wf2, bf2.reshape(1, 1))
    return out


# speedup vs baseline: 18.3536x; 18.3536x over previous
import functools

import jax
import jax.numpy as jnp
from jax import lax
from jax.experimental import pallas as pl
from jax.experimental.pallas import tpu as pltpu
from jax.experimental.pallas import tpu_sc as plsc

N = 10000
D = 128
E = 320000
NC = 2
NS = 16
NW = NC * NS
EPT = E // NW
K = 80
NCHUNK = EPT // K
NPAD = 10240
RPT = NPAD // NS
R = 400
GRID = N // R

_mesh = plsc.VectorSubcoreMesh(core_axis_name="c", subcore_axis_name="s")


@functools.partial(
    pl.kernel,
    out_type=jax.ShapeDtypeStruct((NC, NPAD, D), jnp.float32),
    mesh=_mesh,
    scratch_types=[
        pltpu.VMEM((K, D), jnp.float32),
        pltpu.VMEM((K,), jnp.int32),
        pltpu.VMEM((K,), jnp.int32),
        pltpu.VMEM_SHARED((NPAD, D), jnp.float32),
        pltpu.SemaphoreType.DMA,
        pltpu.SemaphoreType.DMA,
    ],
)
def _deg_kernel(dst_hbm, ones_hbm, zrows_hbm, out_hbm, ones_v, idx0, idx1,
                acc, sem0, sem1):
    cid = lax.axis_index("c")
    sid = lax.axis_index("s")
    wid = sid * NC + cid
    base = wid * EPT
    idxb = (idx0, idx1)
    semb = (sem0, sem1)

    pltpu.sync_copy(ones_hbm, ones_v)
    pltpu.sync_copy(zrows_hbm, acc.at[pl.ds(sid * RPT, RPT)])
    plsc.subcore_barrier()

    def _cp(c, b):
        off = pl.multiple_of(base + c * K, 8)
        return pltpu.make_async_copy(dst_hbm.at[pl.ds(off, K)], idxb[b], semb[b])

    def scatter(b):
        pltpu.sync_copy(ones_v, acc.at[idxb[b]], add=True)

    _cp(0, 0).start()

    def pair(j, carry):
        a = 2 * j
        _cp(a + 1, 1).start()
        _cp(a, 0).wait()
        scatter(0)
        _cp(a + 2, 0).start()
        _cp(a + 1, 1).wait()
        scatter(1)
        return carry

    lax.fori_loop(0, (NCHUNK - 1) // 2, pair, 0)
    _cp(NCHUNK - 1, 0).wait()
    scatter(0)

    plsc.subcore_barrier()
    pltpu.sync_copy(acc.at[pl.ds(sid * RPT, RPT)],
                    out_hbm.at[cid, pl.ds(sid * RPT, RPT)])


@functools.partial(
    pl.kernel,
    out_type=jax.ShapeDtypeStruct((NC, NPAD, D), jnp.float32),
    mesh=_mesh,
    scratch_types=[
        pltpu.VMEM((K,), jnp.int32),
        pltpu.VMEM((K,), jnp.int32),
        pltpu.VMEM((K,), jnp.int32),
        pltpu.VMEM((K,), jnp.int32),
        pltpu.VMEM((K, D), jnp.float32),
        pltpu.VMEM((K, D), jnp.float32),
        pltpu.VMEM_SHARED((NPAD, D), jnp.float32),
        pltpu.SemaphoreType.DMA,
        pltpu.SemaphoreType.DMA,
    ],
)
def _agg_kernel(hp_hbm, src_hbm, dst_hbm, zrows_hbm, out_hbm,
                s0, s1, d0, d1, r0, r1, acc, g0, g1):
    cid = lax.axis_index("c")
    sid = lax.axis_index("s")
    wid = sid * NC + cid
    base = wid * EPT
    sb = (s0, s1)
    db = (d0, d1)
    rb = (r0, r1)
    gsem = (g0, g1)

    pltpu.sync_copy(zrows_hbm, acc.at[pl.ds(sid * RPT, RPT)])
    plsc.subcore_barrier()

    def start(c, b):
        off = pl.multiple_of(base + c * K, 8)
        pltpu.sync_copy(src_hbm.at[pl.ds(off, K)], sb[b])
        pltpu.sync_copy(dst_hbm.at[pl.ds(off, K)], db[b])
        pltpu.make_async_copy(hp_hbm.at[sb[b]], rb[b], gsem[b]).start()

    def drain(b):
        pltpu.make_async_copy(hp_hbm.at[sb[b]], rb[b], gsem[b]).wait()
        pltpu.sync_copy(rb[b], acc.at[db[b]], add=True)

    start(0, 0)

    def pair(j, carry):
        a = 2 * j
        start(a + 1, 1)
        drain(0)
        start(a + 2, 0)
        drain(1)
        return carry

    lax.fori_loop(0, (NCHUNK - 1) // 2, pair, 0)
    drain(0)

    plsc.subcore_barrier()
    pltpu.sync_copy(acc.at[pl.ds(sid * RPT, RPT)],
                    out_hbm.at[cid, pl.ds(sid * RPT, RPT)])


def _dinv_from(deg_blk):
    deg = deg_blk[0, :, 0] + deg_blk[1, :, 0] + 1.0
    return lax.rsqrt(deg)[:, None]


def _tc1_body(x_ref, w1_ref, deg_ref, o_ref):
    dinv = _dinv_from(deg_ref[...])
    h = jnp.dot(x_ref[...], w1_ref[...], preferred_element_type=jnp.float32)
    o_ref[...] = h * dinv


def _tc2_body(agg_ref, hp_ref, deg_ref, b1_ref, w2_ref, o_ref):
    dinv = _dinv_from(deg_ref[...])
    s = agg_ref[0] + agg_ref[1] + hp_ref[...]
    z = jnp.tanh(dinv * s + b1_ref[...])
    o_ref[...] = dinv * jnp.dot(z, w2_ref[...], preferred_element_type=jnp.float32)


def _tc3_body(agg_ref, hp_ref, deg_ref, b2_ref, wf1_ref, bf1_ref, wf2_ref,
              bf2_ref, o_ref):
    dinv = _dinv_from(deg_ref[...])
    s = agg_ref[0] + agg_ref[1] + hp_ref[...]
    z = jnp.tanh(dinv * s + b2_ref[...])
    f = jnp.tanh(jnp.dot(z, wf1_ref[...], preferred_element_type=jnp.float32)
                 + bf1_ref[...])
    o_ref[...] = (jnp.dot(f, wf2_ref[...], preferred_element_type=jnp.float32)
                  + bf2_ref[...])


def _row_spec(width):
    return pl.BlockSpec((R, width), lambda i: (i, 0))


def _full(shape):
    return pl.BlockSpec(shape, lambda i, _s=shape: tuple(0 for _ in _s))


_agg_spec = pl.BlockSpec((NC, R, D), lambda i: (0, i, 0))

_tc1 = pl.pallas_call(
    _tc1_body,
    grid=(GRID,),
    in_specs=[_row_spec(D), _full((D, D)), _agg_spec],
    out_specs=_row_spec(D),
    out_shape=jax.ShapeDtypeStruct((N, D), jnp.float32),
)

_tc2 = pl.pallas_call(
    _tc2_body,
    grid=(GRID,),
    in_specs=[_agg_spec, _row_spec(D), _agg_spec, _full((1, D)),
              _full((D, D))],
    out_specs=_row_spec(D),
    out_shape=jax.ShapeDtypeStruct((N, D), jnp.float32),
)

_tc3 = pl.pallas_call(
    _tc3_body,
    grid=(GRID,),
    in_specs=[_agg_spec, _row_spec(D), _agg_spec, _full((1, D)),
              _full((D, 64)), _full((1, 64)), _full((64, 1)), _full((1, 1))],
    out_specs=_row_spec(1),
    out_shape=jax.ShapeDtypeStruct((N, 1), jnp.float32),
)


def kernel(x, edge_index, W1, b1, W2, b2, Wf1, bf1, Wf2, bf2):
    ei = edge_index.astype(jnp.int32)
    src = ei[0]
    dst = ei[1]
    z128 = jnp.zeros((RPT, D), jnp.float32)

    degp = _deg_kernel(dst, jnp.ones((K, D), jnp.float32), z128)

    h1p = _tc1(x, W1, degp)
    agg1 = _agg_kernel(h1p, src, dst, z128)
    h2p = _tc2(agg1, h1p, degp, b1.reshape(1, D), W2)
    agg2 = _agg_kernel(h2p, src, dst, z128)
    out = _tc3(agg2, h2p, degp, b2.reshape(1, D), Wf1, bf1.reshape(1, 64),
               Wf2, bf2.reshape(1, 1))
    return out


# speedup vs baseline: 26.6068x; 1.4497x over previous
import functools

import jax
import jax.numpy as jnp
from jax import lax
from jax.experimental import pallas as pl
from jax.experimental.pallas import tpu as pltpu
from jax.experimental.pallas import tpu_sc as plsc

N = 10000
D = 128
E = 320000
NC = 2
NS = 16
NW = NC * NS
EPT = E // NW
K = 80
NCHUNK = EPT // K
NPAD = 10240
RPT = NPAD // NS
R = 400
GRID = N // R

_mesh = plsc.VectorSubcoreMesh(core_axis_name="c", subcore_axis_name="s")


NB = 4


@functools.partial(
    pl.kernel,
    out_type=jax.ShapeDtypeStruct((NC, NPAD, D), jnp.float32),
    mesh=_mesh,
    scratch_types=[
        pltpu.VMEM((K, D), jnp.float32),
        [pltpu.VMEM((K,), jnp.int32) for _ in range(NB)],
        pltpu.VMEM_SHARED((NPAD, D), jnp.float32),
        [pltpu.SemaphoreType.DMA for _ in range(NB)],
        [pltpu.SemaphoreType.DMA for _ in range(NB)],
    ],
)
def _deg_kernel(dst_hbm, ones_hbm, zrows_hbm, out_hbm, ones_v, dd, acc,
                di, ss):
    cid = lax.axis_index("c")
    sid = lax.axis_index("s")
    wid = sid * NC + cid
    base = wid * EPT

    pltpu.sync_copy(ones_hbm, ones_v)
    pltpu.sync_copy(zrows_hbm, acc.at[pl.ds(sid * RPT, RPT)])
    plsc.subcore_barrier()

    def icp(c, b):
        off = pl.multiple_of(base + c * K, 8)
        return pltpu.make_async_copy(dst_hbm.at[pl.ds(off, K)], dd[b], di[b])

    def scp(b):
        return pltpu.make_async_copy(ones_v, acc.at[dd[b]], ss[b])

    def sstart(b):
        pltpu.async_copy(ones_v, acc.at[dd[b]], ss[b], add=True)

    icp(0, 0).start()
    icp(1, 1).start()
    icp(0, 0).wait()
    sstart(0)
    icp(2, 2).start()
    icp(1, 1).wait()
    sstart(1)
    icp(3, 3).start()

    def body(c, b):
        icp(c, b).wait()
        sstart(b)
        scp((b + 2) % NB).wait()
        icp(c + 2, (b + 2) % NB).start()

    def group(j, carry):
        for b2 in range(NB):
            body(j * NB + 2 + b2, (b2 + 2) % NB)
        return carry

    lax.fori_loop(0, (NCHUNK - 4) // NB, group, 0)
    c0 = 2 + NB * ((NCHUNK - 4) // NB)
    body(c0, c0 % NB)
    for c in (c0 + 1, c0 + 2):
        b = c % NB
        icp(c, b).wait()
        sstart(b)
        scp((b + 2) % NB).wait()
    scp((c0 + 1) % NB).wait()
    scp((c0 + 2) % NB).wait()

    plsc.subcore_barrier()
    pltpu.sync_copy(acc.at[pl.ds(sid * RPT, RPT)],
                    out_hbm.at[cid, pl.ds(sid * RPT, RPT)])


@functools.partial(
    pl.kernel,
    out_type=jax.ShapeDtypeStruct((NC, NPAD, D), jnp.float32),
    mesh=_mesh,
    scratch_types=[
        [pltpu.VMEM((K,), jnp.int32) for _ in range(NB)],
        [pltpu.VMEM((K,), jnp.int32) for _ in range(NB)],
        [pltpu.VMEM((K, D), jnp.float32) for _ in range(NB)],
        pltpu.VMEM_SHARED((NPAD, D), jnp.float32),
        [pltpu.SemaphoreType.DMA for _ in range(NB)],
        [pltpu.SemaphoreType.DMA for _ in range(NB)],
        [pltpu.SemaphoreType.DMA for _ in range(NB)],
    ],
)
def _agg_kernel(hp_hbm, src_hbm, dst_hbm, zrows_hbm, out_hbm,
                sb, db, rb, acc, si, di, gsem):
    cid = lax.axis_index("c")
    sid = lax.axis_index("s")
    wid = sid * NC + cid
    base = wid * EPT

    pltpu.sync_copy(zrows_hbm, acc.at[pl.ds(sid * RPT, RPT)])
    plsc.subcore_barrier()

    def icp(c, b):
        off = pl.multiple_of(base + c * K, 8)
        return (pltpu.make_async_copy(src_hbm.at[pl.ds(off, K)], sb[b], si[b]),
                pltpu.make_async_copy(dst_hbm.at[pl.ds(off, K)], db[b], di[b]))

    def istart(c, b):
        s, d2 = icp(c, b)
        s.start()
        d2.start()

    def iwait(c, b):
        s, d2 = icp(c, b)
        s.wait()
        d2.wait()

    def gcp(b):
        return pltpu.make_async_copy(hp_hbm.at[sb[b]], rb[b], gsem[b])

    def scatter(b):
        pltpu.sync_copy(rb[b], acc.at[db[b]], add=True)

    for b in range(NB):
        istart(b, b)
    iwait(0, 0)
    gcp(0).start()
    iwait(1, 1)
    gcp(1).start()

    def body(c, b):
        gcp(b).wait()
        iwait(c + 2, (b + 2) % NB)
        gcp((b + 2) % NB).start()
        scatter(b)
        istart(c + 4, b)

    def group(j, carry):
        for b2 in range(NB):
            body(j * NB + b2, b2)
        return carry

    lax.fori_loop(0, (NCHUNK - 5) // NB, group, 0)
    for c in range(NB * ((NCHUNK - 5) // NB), NCHUNK):
        b = c % NB
        gcp(b).wait()
        if c + 2 < NCHUNK:
            iwait(c + 2, (b + 2) % NB)
            gcp((b + 2) % NB).start()
        scatter(b)
        if c + 4 < NCHUNK:
            istart(c + 4, b)

    plsc.subcore_barrier()
    pltpu.sync_copy(acc.at[pl.ds(sid * RPT, RPT)],
                    out_hbm.at[cid, pl.ds(sid * RPT, RPT)])


def _dinv_from(deg_blk):
    deg = deg_blk[0, :, 0] + deg_blk[1, :, 0] + 1.0
    return lax.rsqrt(deg)[:, None]


def _tc1_body(x_ref, w1_ref, deg_ref, o_ref):
    dinv = _dinv_from(deg_ref[...])
    h = jnp.dot(x_ref[...], w1_ref[...], preferred_element_type=jnp.float32)
    o_ref[...] = h * dinv


def _tc2_body(agg_ref, hp_ref, deg_ref, b1_ref, w2_ref, o_ref):
    dinv = _dinv_from(deg_ref[...])
    s = agg_ref[0] + agg_ref[1] + hp_ref[...]
    z = jnp.tanh(dinv * s + b1_ref[...])
    o_ref[...] = dinv * jnp.dot(z, w2_ref[...], preferred_element_type=jnp.float32)


def _tc3_body(agg_ref, hp_ref, deg_ref, b2_ref, wf1_ref, bf1_ref, wf2_ref,
              bf2_ref, o_ref):
    dinv = _dinv_from(deg_ref[...])
    s = agg_ref[0] + agg_ref[1] + hp_ref[...]
    z = jnp.tanh(dinv * s + b2_ref[...])
    f = jnp.tanh(jnp.dot(z, wf1_ref[...], preferred_element_type=jnp.float32)
                 + bf1_ref[...])
    o_ref[...] = (jnp.dot(f, wf2_ref[...], preferred_element_type=jnp.float32)
                  + bf2_ref[...])


def _row_spec(width):
    return pl.BlockSpec((R, width), lambda i: (i, 0))


def _full(shape):
    return pl.BlockSpec(shape, lambda i, _s=shape: tuple(0 for _ in _s))


_agg_spec = pl.BlockSpec((NC, R, D), lambda i: (0, i, 0))

_tc1 = pl.pallas_call(
    _tc1_body,
    grid=(GRID,),
    in_specs=[_row_spec(D), _full((D, D)), _agg_spec],
    out_specs=_row_spec(D),
    out_shape=jax.ShapeDtypeStruct((N, D), jnp.float32),
)

_tc2 = pl.pallas_call(
    _tc2_body,
    grid=(GRID,),
    in_specs=[_agg_spec, _row_spec(D), _agg_spec, _full((1, D)),
              _full((D, D))],
    out_specs=_row_spec(D),
    out_shape=jax.ShapeDtypeStruct((N, D), jnp.float32),
)

_tc3 = pl.pallas_call(
    _tc3_body,
    grid=(GRID,),
    in_specs=[_agg_spec, _row_spec(D), _agg_spec, _full((1, D)),
              _full((D, 64)), _full((1, 64)), _full((64, 1)), _full((1, 1))],
    out_specs=_row_spec(1),
    out_shape=jax.ShapeDtypeStruct((N, 1), jnp.float32),
)


def kernel(x, edge_index, W1, b1, W2, b2, Wf1, bf1, Wf2, bf2):
    ei = edge_index.astype(jnp.int32)
    src = ei[0]
    dst = ei[1]
    z128 = jnp.zeros((RPT, D), jnp.float32)

    degp = _deg_kernel(dst, jnp.ones((K, D), jnp.float32), z128)

    h1p = _tc1(x, W1, degp)
    agg1 = _agg_kernel(h1p, src, dst, z128)
    h2p = _tc2(agg1, h1p, degp, b1.reshape(1, D), W2)
    agg2 = _agg_kernel(h2p, src, dst, z128)
    out = _tc3(agg2, h2p, degp, b2.reshape(1, D), Wf1, bf1.reshape(1, 64),
               Wf2, bf2.reshape(1, 1))
    return out
